# baseline (speedup 1.0000x reference)
import functools

import jax
import jax.numpy as jnp
from jax import lax
from jax.experimental import pallas as pl
from jax.experimental.pallas import tpu as pltpu
from jax.experimental.pallas import tpu_sc as plsc

NXc, NYc, Cc, Bc, Pc = 512, 512, 64, 4, 40000
CPT = 32768
LOG2_CPT = 15
NREG = 64
RCELLS = NXc
CHUNK = 2000
NCHUNK = Pc // CHUNK
DUMP = CPT
GROWS = 64


def _iota16():
    return lax.iota(jnp.int32, 16)


def _popcount(mask):
    return jnp.sum(mask.astype(jnp.int32))


def _sload(ref, i):
    return ref[pl.ds(i, 16)][0]


def _sstore(ref, i, v):
    plsc.store_scatter(ref, [jnp.full((16,), i, jnp.int32)],
                       jnp.full((16,), v, jnp.int32), mask=_iota16() == 0)


def _body(f128_hbm, b_hbm, y_hbm, x_hbm, out_hbm,
          cbuf, rawl, wmap, staging, gbuf0, gbuf1, idxb0, idxb1,
          sem0, sem1, boff, pos):
    wid = lax.axis_index("s") * 2 + lax.axis_index("c")

    neg1 = jnp.full((16,), -1, jnp.int32)

    @pl.loop(0, CPT, step=16)
    def _(i):
        wmap[pl.ds(i, 16)] = neg1

    def a1_chunk(ci, count):
        base = ci * CHUNK
        pltpu.sync_copy(b_hbm.at[pl.ds(base, CHUNK)],
                        cbuf.at[pl.ds(0, CHUNK)])
        pltpu.sync_copy(y_hbm.at[pl.ds(base, CHUNK)],
                        cbuf.at[pl.ds(CHUNK, CHUNK)])
        pltpu.sync_copy(x_hbm.at[pl.ds(base, CHUNK)],
                        cbuf.at[pl.ds(2 * CHUNK, CHUNK)])

        def vloop(vi, cnt):
            off = vi * 16
            bv = cbuf[pl.ds(off, 16)]
            yv = cbuf[pl.ds(CHUNK + off, 16)]
            xv = cbuf[pl.ds(2 * CHUNK + off, 16)]
            cell = bv * (NYc * NXc) + yv * NXc + xv
            valid = lax.shift_right_logical(cell, LOG2_CPT) == wid
            local = lax.bitwise_and(cell, CPT - 1)
            p = base + off + _iota16()

            wv = plsc.load_gather(wmap, [local])
            m0 = valid & (p > wv)

            def wcond(m):
                return jnp.any(m)

            def wbody(m):
                plsc.store_scatter(wmap, [local], p, mask=m)
                w2 = plsc.load_gather(wmap, [local])
                return valid & (p > w2)

            lax.while_loop(wcond, wbody, m0)

            packed = lax.bitwise_or(local, lax.shift_left(p, LOG2_CPT))
            plsc.store_compressed(rawl.at[pl.ds(cnt, 16)], packed, mask=valid)
            return cnt + _popcount(valid)

        return lax.fori_loop(0, CHUNK // 16, vloop, count)

    with jax.named_scope("phaseA1"):
        na = lax.fori_loop(0, NCHUNK, a1_chunk, jnp.int32(0))

    @pl.loop(0, NREG)
    def _(r):
        pos[r] = 0

    def count_body(e, _):
        pk = _sload(rawl, e)
        local = lax.bitwise_and(pk, CPT - 1)
        p = lax.shift_right_logical(pk, LOG2_CPT)
        live_i = (_sload(wmap, local) == p).astype(jnp.int32)
        rg = lax.shift_right_logical(local, 9)
        pos[rg] += live_i
        _sstore(rawl, e, lax.bitwise_or(pk, lax.shift_left(live_i, 31)))
        return 0

    with jax.named_scope("phaseA2count"):
        lax.fori_loop(0, na, count_body, 0)

    def prefix_body(r, acc):
        c = pos[r]
        boff[r] = acc
        pos[r] = acc
        return acc + c

    nb = lax.fori_loop(0, NREG, prefix_body, jnp.int32(0))
    boff[NREG] = nb

    def place_body(e, _):
        pk = _sload(rawl, e)
        live = pk < 0
        pkc = lax.bitwise_and(pk, 0x7FFFFFFF)
        rg = lax.shift_right_logical(lax.bitwise_and(pkc, CPT - 1), 9)
        o = pos[rg]
        dest = jnp.where(live, o, DUMP)
        _sstore(wmap, dest, pkc)
        pos[rg] = o + live.astype(jnp.int32)
        return 0

    with jax.named_scope("phaseA2place"):
        lax.fori_loop(0, na, place_body, 0)

    zero16 = jnp.zeros((16,), jnp.float32)
    chanbase = (wid >> 3) * Cc
    yrowbase = lax.bitwise_and(wid, 7) * NREG

    @pl.loop(0, Cc)
    def _(c):
        @pl.loop(0, RCELLS, step=16)
        def _(j):
            staging[c, pl.ds(j, 16)] = zero16

    def fire(r, idx_s, gbuf_s, sem_s):
        @pl.when(r < NREG)
        def _():
            start = boff[r]
            end = boff[r + 1]
            for v in range(GROWS // 16):
                pk = wmap[pl.ds(start + v * 16, 16)]
                okm = (start + v * 16 + _iota16()) < end
                rid = jnp.where(okm,
                                lax.shift_right_logical(pk, LOG2_CPT + 1), 0)
                idx_s[pl.ds(v * 16, 16)] = rid
            pltpu.async_copy(f128_hbm.at[idx_s], gbuf_s, sem_s)

    def place_range(gbuf_s, lstart, n):
        def place(j, _):
            pkj = _sload(wmap, lstart + j)
            lr = lax.bitwise_and(pkj, RCELLS - 1)
            pj = lax.shift_right_logical(pkj, LOG2_CPT)
            half = lax.bitwise_and(pj, 1) * Cc
            lr_s = jnp.full((16,), lr, jnp.int32)
            for q in range(4):
                vals = gbuf_s[j, pl.ds(half + q * 16, 16)]
                plsc.store_scatter(staging, [q * 16 + _iota16(), lr_s], vals)
            return 0

        lax.fori_loop(0, n, place, 0)

    def process(r, idx_s, gbuf_s, sem_s):
        start = boff[r]
        end = boff[r + 1]
        n = end - start
        pltpu.make_async_copy(f128_hbm.at[idx_s], gbuf_s, sem_s).wait()
        place_range(gbuf_s.at[pl.ds(0, GROWS)], start, jnp.minimum(n, GROWS))

        @pl.when(n > GROWS)
        def _():
            def chunk_body(ch, _):
                cstart = start + GROWS + ch * 16
                pk = wmap[pl.ds(cstart, 16)]
                okm = (cstart + _iota16()) < end
                rid = jnp.where(okm,
                                lax.shift_right_logical(pk, LOG2_CPT + 1), 0)
                idx_s[pl.ds(0, 16)] = rid
                pltpu.sync_copy(f128_hbm.at[idx_s.at[pl.ds(0, 16)]],
                                gbuf_s.at[pl.ds(0, 16)])
                k = jnp.minimum(jnp.int32(16), end - cstart)
                place_range(gbuf_s.at[pl.ds(0, 16)], cstart, k)
                return 0

            lax.fori_loop(0, (n - GROWS + 15) >> 4, chunk_body, 0)

        pltpu.sync_copy(staging,
                        out_hbm.at[pl.ds(chanbase, Cc),
                                   yrowbase + r,
                                   pl.ds(0, NXc)])

        def clean(e, _):
            pkj = _sload(wmap, e)
            lr = lax.bitwise_and(pkj, RCELLS - 1)
            lr_s = jnp.full((16,), lr, jnp.int32)
            for q in range(4):
                plsc.store_scatter(staging, [q * 16 + _iota16(), lr_s], zero16)
            return 0

        lax.fori_loop(start, end, clean, 0)

    with jax.named_scope("phaseB"):
        fire(jnp.int32(0), idxb0, gbuf0, sem0)

        def pair_body(rr, _):
            r0 = rr * 2
            fire(r0 + 1, idxb1, gbuf1, sem1)
            process(r0, idxb0, gbuf0, sem0)
            fire(r0 + 2, idxb0, gbuf0, sem0)
            process(r0 + 1, idxb1, gbuf1, sem1)
            return 0

        lax.fori_loop(0, NREG // 2, pair_body, 0)


@jax.jit
def kernel(pillar_features, voxel_coords):
    f128 = pillar_features.reshape(Pc // 2, 2 * Cc)
    cols = voxel_coords.T
    bcol = cols[0]
    ycol = cols[2]
    xcol = cols[3]

    mesh = plsc.VectorSubcoreMesh(core_axis_name="c", subcore_axis_name="s")
    run = pl.kernel(
        _body,
        out_type=jax.ShapeDtypeStruct((Bc * Cc, NYc, NXc), jnp.float32),
        mesh=mesh,
        scratch_types=[
            pltpu.VMEM((3 * CHUNK,), jnp.int32),
            pltpu.VMEM((Pc,), jnp.int32),
            pltpu.VMEM((CPT + 32,), jnp.int32),
            pltpu.VMEM((Cc, RCELLS), jnp.float32),
            pltpu.VMEM((GROWS, 2 * Cc), jnp.float32),
            pltpu.VMEM((GROWS, 2 * Cc), jnp.float32),
            pltpu.VMEM((GROWS,), jnp.int32),
            pltpu.VMEM((GROWS,), jnp.int32),
            pltpu.SemaphoreType.DMA,
            pltpu.SemaphoreType.DMA,
            pltpu.SMEM((NREG + 1,), jnp.int32),
            pltpu.SMEM((NREG,), jnp.int32),
        ],
        compiler_params=pltpu.CompilerParams(needs_layout_passes=False),
    )
    out_flat = run(f128, bcol, ycol, xcol)
    return out_flat.reshape(Bc, Cc, NYc, NXc)


# speedup vs baseline: 1.0000x; 1.0000x over previous
import functools

import jax
import jax.numpy as jnp
from jax import lax
from jax.experimental import pallas as pl
from jax.experimental.pallas import tpu as pltpu
from jax.experimental.pallas import tpu_sc as plsc

NXc, NYc, Cc, Bc, Pc = 512, 512, 64, 4, 40000
CPT = 32768
LOG2_CPT = 15
NREG = 64
RCELLS = NXc
CHUNK = 2000
NCHUNK = Pc // CHUNK
DUMP = CPT
GROWS = 64


def _iota16():
    return lax.iota(jnp.int32, 16)


def _popcount(mask):
    return jnp.sum(mask.astype(jnp.int32))


def _sload(ref, i):
    return ref[pl.ds(i, 16)][0]


def _sstore(ref, i, v):
    plsc.store_scatter(ref, [jnp.full((16,), i, jnp.int32)],
                       jnp.full((16,), v, jnp.int32), mask=_iota16() == 0)


def _body(f128_hbm, b_hbm, y_hbm, x_hbm, out_hbm,
          cbuf, rawl, wmap, staging, gbuf0, gbuf1, idxb0, idxb1,
          sem0, sem1, boff, pos):
    wid = lax.axis_index("s") * 2 + lax.axis_index("c")

    neg1 = jnp.full((16,), -1, jnp.int32)

    @pl.loop(0, CPT, step=16)
    def _(i):
        wmap[pl.ds(i, 16)] = neg1

    def a1_chunk(ci, count):
        base = ci * CHUNK
        pltpu.sync_copy(b_hbm.at[pl.ds(base, CHUNK)],
                        cbuf.at[pl.ds(0, CHUNK)])
        pltpu.sync_copy(y_hbm.at[pl.ds(base, CHUNK)],
                        cbuf.at[pl.ds(CHUNK, CHUNK)])
        pltpu.sync_copy(x_hbm.at[pl.ds(base, CHUNK)],
                        cbuf.at[pl.ds(2 * CHUNK, CHUNK)])

        def vloop(vi, cnt):
            off = vi * 16
            bv = cbuf[pl.ds(off, 16)]
            yv = cbuf[pl.ds(CHUNK + off, 16)]
            xv = cbuf[pl.ds(2 * CHUNK + off, 16)]
            cell = bv * (NYc * NXc) + yv * NXc + xv
            valid = lax.shift_right_logical(cell, LOG2_CPT) == wid
            local = lax.bitwise_and(cell, CPT - 1)
            p = base + off + _iota16()

            wv = plsc.load_gather(wmap, [local])
            m0 = valid & (p > wv)

            def wcond(m):
                return jnp.any(m)

            def wbody(m):
                plsc.store_scatter(wmap, [local], p, mask=m)
                w2 = plsc.load_gather(wmap, [local])
                return valid & (p > w2)

            lax.while_loop(wcond, wbody, m0)

            packed = lax.bitwise_or(local, lax.shift_left(p, LOG2_CPT))
            plsc.store_compressed(rawl.at[pl.ds(cnt, 16)], packed, mask=valid)
            return cnt + _popcount(valid)

        return lax.fori_loop(0, CHUNK // 16, vloop, count)

    with jax.named_scope("phaseA1"):
        na = lax.fori_loop(0, NCHUNK, a1_chunk, jnp.int32(0))

    @pl.loop(0, NREG)
    def _(r):
        pos[r] = 0

    def count_body(e, _):
        pk = _sload(rawl, e)
        local = lax.bitwise_and(pk, CPT - 1)
        p = lax.shift_right_logical(pk, LOG2_CPT)
        live_i = (_sload(wmap, local) == p).astype(jnp.int32)
        rg = lax.shift_right_logical(local, 9)
        pos[rg] += live_i
        _sstore(rawl, e, lax.bitwise_or(pk, lax.shift_left(live_i, 31)))
        return 0

    with jax.named_scope("phaseA2count"):
        lax.fori_loop(0, na, count_body, 0)

    def prefix_body(r, acc):
        c = pos[r]
        boff[r] = acc
        pos[r] = acc
        return acc + c

    nb = lax.fori_loop(0, NREG, prefix_body, jnp.int32(0))
    boff[NREG] = nb

    def place_body(e, _):
        pk = _sload(rawl, e)
        live = pk < 0
        pkc = lax.bitwise_and(pk, 0x7FFFFFFF)
        rg = lax.shift_right_logical(lax.bitwise_and(pkc, CPT - 1), 9)
        o = pos[rg]
        dest = jnp.where(live, o, DUMP)
        _sstore(wmap, dest, pkc)
        pos[rg] = o + live.astype(jnp.int32)
        return 0

    with jax.named_scope("phaseA2place"):
        lax.fori_loop(0, na, place_body, 0)

    zero16 = jnp.zeros((16,), jnp.float32)
    chanbase = (wid >> 3) * Cc
    yrowbase = lax.bitwise_and(wid, 7) * NREG

    @pl.loop(0, Cc)
    def _(c):
        @pl.loop(0, RCELLS, step=16)
        def _(j):
            staging[c, pl.ds(j, 16)] = zero16

    def fire(r, idx_s, gbuf_s, sem_s):
        @pl.when(r < NREG)
        def _():
            start = boff[r]
            end = boff[r + 1]
            for v in range(GROWS // 16):
                pk = wmap[pl.ds(start + v * 16, 16)]
                okm = (start + v * 16 + _iota16()) < end
                rid = jnp.where(okm,
                                lax.shift_right_logical(pk, LOG2_CPT + 1), 0)
                idx_s[pl.ds(v * 16, 16)] = rid
            pltpu.sync_copy(f128_hbm.at[idx_s], gbuf_s)

    def place_range(gbuf_s, lstart, n):
        def place(j, _):
            pkj = _sload(wmap, lstart + j)
            lr = lax.bitwise_and(pkj, RCELLS - 1)
            pj = lax.shift_right_logical(pkj, LOG2_CPT)
            half = lax.bitwise_and(pj, 1) * Cc
            lr_s = jnp.full((16,), lr, jnp.int32)
            for q in range(4):
                vals = gbuf_s[j, pl.ds(half + q * 16, 16)]
                plsc.store_scatter(staging, [q * 16 + _iota16(), lr_s], vals)
            return 0

        lax.fori_loop(0, n, place, 0)

    def process(r, idx_s, gbuf_s, sem_s):
        start = boff[r]
        end = boff[r + 1]
        n = end - start
        place_range(gbuf_s.at[pl.ds(0, GROWS)], start, jnp.minimum(n, GROWS))

        @pl.when(n > GROWS)
        def _():
            def chunk_body(ch, _):
                cstart = start + GROWS + ch * 16
                pk = wmap[pl.ds(cstart, 16)]
                okm = (cstart + _iota16()) < end
                rid = jnp.where(okm,
                                lax.shift_right_logical(pk, LOG2_CPT + 1), 0)
                idx_s[pl.ds(0, 16)] = rid
                pltpu.sync_copy(f128_hbm.at[idx_s.at[pl.ds(0, 16)]],
                                gbuf_s.at[pl.ds(0, 16)])
                k = jnp.minimum(jnp.int32(16), end - cstart)
                place_range(gbuf_s.at[pl.ds(0, 16)], cstart, k)
                return 0

            lax.fori_loop(0, (n - GROWS + 15) >> 4, chunk_body, 0)

        pltpu.sync_copy(staging,
                        out_hbm.at[pl.ds(chanbase, Cc),
                                   yrowbase + r,
                                   pl.ds(0, NXc)])

        def clean(e, _):
            pkj = _sload(wmap, e)
            lr = lax.bitwise_and(pkj, RCELLS - 1)
            lr_s = jnp.full((16,), lr, jnp.int32)
            for q in range(4):
                plsc.store_scatter(staging, [q * 16 + _iota16(), lr_s], zero16)
            return 0

        lax.fori_loop(start, end, clean, 0)

    with jax.named_scope("phaseB"):
        fire(jnp.int32(0), idxb0, gbuf0, sem0)

        def pair_body(rr, _):
            r0 = rr * 2
            fire(r0 + 1, idxb1, gbuf1, sem1)
            process(r0, idxb0, gbuf0, sem0)
            fire(r0 + 2, idxb0, gbuf0, sem0)
            process(r0 + 1, idxb1, gbuf1, sem1)
            return 0

        lax.fori_loop(0, NREG // 2, pair_body, 0)


@jax.jit
def kernel(pillar_features, voxel_coords):
    f128 = pillar_features.reshape(Pc // 2, 2 * Cc)
    cols = voxel_coords.T
    bcol = cols[0]
    ycol = cols[2]
    xcol = cols[3]

    mesh = plsc.VectorSubcoreMesh(core_axis_name="c", subcore_axis_name="s")
    run = pl.kernel(
        _body,
        out_type=jax.ShapeDtypeStruct((Bc * Cc, NYc, NXc), jnp.float32),
        mesh=mesh,
        scratch_types=[
            pltpu.VMEM((3 * CHUNK,), jnp.int32),
            pltpu.VMEM((Pc,), jnp.int32),
            pltpu.VMEM((CPT + 32,), jnp.int32),
            pltpu.VMEM((Cc, RCELLS), jnp.float32),
            pltpu.VMEM((GROWS, 2 * Cc), jnp.float32),
            pltpu.VMEM((GROWS, 2 * Cc), jnp.float32),
            pltpu.VMEM((GROWS,), jnp.int32),
            pltpu.VMEM((GROWS,), jnp.int32),
            pltpu.SemaphoreType.DMA,
            pltpu.SemaphoreType.DMA,
            pltpu.SMEM((NREG + 1,), jnp.int32),
            pltpu.SMEM((NREG,), jnp.int32),
        ],
        compiler_params=pltpu.CompilerParams(needs_layout_passes=False),
    )
    out_flat = run(f128, bcol, ycol, xcol)
    return out_flat.reshape(Bc, Cc, NYc, NXc)


# speedup vs baseline: 6.7743x; 6.7741x over previous
import functools

import jax
import jax.numpy as jnp
from jax import lax
from jax.experimental import pallas as pl
from jax.experimental.pallas import tpu as pltpu
from jax.experimental.pallas import tpu_sc as plsc

NXc, NYc, Cc, Bc, Pc = 512, 512, 64, 4, 40000
CPT = 32768
LOG2_CPT = 15
NREG = 64
RCELLS = NXc
CHUNK = 2000
NCHUNK = Pc // CHUNK
DUMP = CPT
GROWS = 64


def _iota16():
    return lax.iota(jnp.int32, 16)


def _popcount(mask):
    return jnp.sum(mask.astype(jnp.int32))


def _sload(ref, i):
    return ref[pl.ds(i, 16)][0]


def _sstore(ref, i, v):
    plsc.store_scatter(ref, [jnp.full((16,), i, jnp.int32)],
                       jnp.full((16,), v, jnp.int32), mask=_iota16() == 0)


def _body(f128_hbm, b_hbm, y_hbm, x_hbm, out_hbm,
          cbuf, rawl, wmap, staging, gbuf0, gbuf1, idxb0, idxb1,
          sem0, sem1, boff, pos):
    wid = lax.axis_index("s") * 2 + lax.axis_index("c")

    neg1 = jnp.full((16,), -1, jnp.int32)

    @pl.loop(0, CPT, step=16)
    def _(i):
        wmap[pl.ds(i, 16)] = neg1

    def a1_chunk(ci, count):
        base = ci * CHUNK
        pltpu.sync_copy(b_hbm.at[pl.ds(base, CHUNK)],
                        cbuf.at[pl.ds(0, CHUNK)])
        pltpu.sync_copy(y_hbm.at[pl.ds(base, CHUNK)],
                        cbuf.at[pl.ds(CHUNK, CHUNK)])
        pltpu.sync_copy(x_hbm.at[pl.ds(base, CHUNK)],
                        cbuf.at[pl.ds(2 * CHUNK, CHUNK)])

        def vloop(vi, cnt):
            off = vi * 16
            bv = cbuf[pl.ds(off, 16)]
            yv = cbuf[pl.ds(CHUNK + off, 16)]
            xv = cbuf[pl.ds(2 * CHUNK + off, 16)]
            cell = bv * (NYc * NXc) + yv * NXc + xv
            valid = lax.shift_right_logical(cell, LOG2_CPT) == wid
            local = lax.bitwise_and(cell, CPT - 1)
            p = base + off + _iota16()

            wv = plsc.load_gather(wmap, [local])
            m0 = valid & (p > wv)

            def wcond(m):
                return jnp.any(m)

            def wbody(m):
                plsc.store_scatter(wmap, [local], p, mask=m)
                w2 = plsc.load_gather(wmap, [local])
                return valid & (p > w2)

            lax.while_loop(wcond, wbody, m0)

            packed = lax.bitwise_or(local, lax.shift_left(p, LOG2_CPT))
            plsc.store_compressed(rawl.at[pl.ds(cnt, 16)], packed, mask=valid)
            return cnt + _popcount(valid)

        return lax.fori_loop(0, CHUNK // 16, vloop, count)

    with jax.named_scope("phaseA1"):
        na = lax.fori_loop(0, NCHUNK, a1_chunk, jnp.int32(0))

    @pl.loop(0, NREG)
    def _(r):
        pos[r] = 0

    def count_body(e, _):
        pk = _sload(rawl, e)
        local = lax.bitwise_and(pk, CPT - 1)
        p = lax.shift_right_logical(pk, LOG2_CPT)
        live_i = (_sload(wmap, local) == p).astype(jnp.int32)
        rg = lax.shift_right_logical(local, 9)
        pos[rg] += live_i
        _sstore(rawl, e, lax.bitwise_or(pk, lax.shift_left(live_i, 31)))
        return 0

    with jax.named_scope("phaseA2count"):
        lax.fori_loop(0, na, count_body, 0)

    def prefix_body(r, acc):
        c = pos[r]
        boff[r] = acc
        pos[r] = acc
        return acc + c

    nb = lax.fori_loop(0, NREG, prefix_body, jnp.int32(0))
    boff[NREG] = nb

    def place_body(e, _):
        pk = _sload(rawl, e)
        live = pk < 0
        pkc = lax.bitwise_and(pk, 0x7FFFFFFF)
        rg = lax.shift_right_logical(lax.bitwise_and(pkc, CPT - 1), 9)
        o = pos[rg]
        dest = jnp.where(live, o, DUMP)
        _sstore(wmap, dest, pkc)
        pos[rg] = o + live.astype(jnp.int32)
        return 0

    with jax.named_scope("phaseA2place"):
        lax.fori_loop(0, na, place_body, 0)

    zero16 = jnp.zeros((16,), jnp.float32)
    chanbase = (wid >> 3) * Cc
    yrowbase = lax.bitwise_and(wid, 7) * NREG

    @pl.loop(0, Cc)
    def _(c):
        @pl.loop(0, RCELLS, step=16)
        def _(j):
            staging[c, pl.ds(j, 16)] = zero16

    def fire(r, idx_s, gbuf_s, sem_s):
        @pl.when(r < NREG)
        def _():
            start = boff[r]
            end = boff[r + 1]
            for v in range(GROWS // 16):
                pk = wmap[pl.ds(start + v * 16, 16)]
                okm = (start + v * 16 + _iota16()) < end
                alt = lax.rem(start + v * 16 + _iota16(), Pc // 2)
                rid = jnp.where(okm,
                                lax.shift_right_logical(pk, LOG2_CPT + 1), alt)
                idx_s[pl.ds(v * 16, 16)] = rid
            pltpu.sync_copy(f128_hbm.at[idx_s], gbuf_s)

    def place_range(gbuf_s, lstart, n):
        def place(j, _):
            pkj = _sload(wmap, lstart + j)
            lr = lax.bitwise_and(pkj, RCELLS - 1)
            pj = lax.shift_right_logical(pkj, LOG2_CPT)
            half = lax.bitwise_and(pj, 1) * Cc
            lr_s = jnp.full((16,), lr, jnp.int32)
            for q in range(4):
                vals = gbuf_s[j, pl.ds(half + q * 16, 16)]
                plsc.store_scatter(staging, [q * 16 + _iota16(), lr_s], vals)
            return 0

        lax.fori_loop(0, n, place, 0)

    def process(r, idx_s, gbuf_s, sem_s):
        start = boff[r]
        end = boff[r + 1]
        n = end - start
        place_range(gbuf_s.at[pl.ds(0, GROWS)], start, jnp.minimum(n, GROWS))

        @pl.when(n > GROWS)
        def _():
            def chunk_body(ch, _):
                cstart = start + GROWS + ch * 16
                pk = wmap[pl.ds(cstart, 16)]
                okm = (cstart + _iota16()) < end
                rid = jnp.where(okm,
                                lax.shift_right_logical(pk, LOG2_CPT + 1), 0)
                idx_s[pl.ds(0, 16)] = rid
                pltpu.sync_copy(f128_hbm.at[idx_s.at[pl.ds(0, 16)]],
                                gbuf_s.at[pl.ds(0, 16)])
                k = jnp.minimum(jnp.int32(16), end - cstart)
                place_range(gbuf_s.at[pl.ds(0, 16)], cstart, k)
                return 0

            lax.fori_loop(0, (n - GROWS + 15) >> 4, chunk_body, 0)

        pltpu.sync_copy(staging,
                        out_hbm.at[pl.ds(chanbase, Cc),
                                   yrowbase + r,
                                   pl.ds(0, NXc)])

        def clean(e, _):
            pkj = _sload(wmap, e)
            lr = lax.bitwise_and(pkj, RCELLS - 1)
            lr_s = jnp.full((16,), lr, jnp.int32)
            for q in range(4):
                plsc.store_scatter(staging, [q * 16 + _iota16(), lr_s], zero16)
            return 0

        lax.fori_loop(start, end, clean, 0)

    with jax.named_scope("phaseB"):
        fire(jnp.int32(0), idxb0, gbuf0, sem0)

        def pair_body(rr, _):
            r0 = rr * 2
            fire(r0 + 1, idxb1, gbuf1, sem1)
            process(r0, idxb0, gbuf0, sem0)
            fire(r0 + 2, idxb0, gbuf0, sem0)
            process(r0 + 1, idxb1, gbuf1, sem1)
            return 0

        lax.fori_loop(0, NREG // 2, pair_body, 0)


@jax.jit
def kernel(pillar_features, voxel_coords):
    f128 = pillar_features.reshape(Pc // 2, 2 * Cc)
    cols = voxel_coords.T
    bcol = cols[0]
    ycol = cols[2]
    xcol = cols[3]

    mesh = plsc.VectorSubcoreMesh(core_axis_name="c", subcore_axis_name="s")
    run = pl.kernel(
        _body,
        out_type=jax.ShapeDtypeStruct((Bc * Cc, NYc, NXc), jnp.float32),
        mesh=mesh,
        scratch_types=[
            pltpu.VMEM((3 * CHUNK,), jnp.int32),
            pltpu.VMEM((Pc,), jnp.int32),
            pltpu.VMEM((CPT + 32,), jnp.int32),
            pltpu.VMEM((Cc, RCELLS), jnp.float32),
            pltpu.VMEM((GROWS, 2 * Cc), jnp.float32),
            pltpu.VMEM((GROWS, 2 * Cc), jnp.float32),
            pltpu.VMEM((GROWS,), jnp.int32),
            pltpu.VMEM((GROWS,), jnp.int32),
            pltpu.SemaphoreType.DMA,
            pltpu.SemaphoreType.DMA,
            pltpu.SMEM((NREG + 1,), jnp.int32),
            pltpu.SMEM((NREG,), jnp.int32),
        ],
        compiler_params=pltpu.CompilerParams(needs_layout_passes=False),
    )
    out_flat = run(f128, bcol, ycol, xcol)
    return out_flat.reshape(Bc, Cc, NYc, NXc)


# speedup vs baseline: 7.0171x; 1.0358x over previous
import functools

import jax
import jax.numpy as jnp
from jax import lax
from jax.experimental import pallas as pl
from jax.experimental.pallas import tpu as pltpu
from jax.experimental.pallas import tpu_sc as plsc

NXc, NYc, Cc, Bc, Pc = 512, 512, 64, 4, 40000
CPT = 32768
LOG2_CPT = 15
NREG = 64
RCELLS = NXc
CHUNK = 2000
NCHUNK = Pc // CHUNK
DUMP = CPT
GROWS = 64


def _iota16():
    return lax.iota(jnp.int32, 16)


def _popcount(mask):
    return plsc.all_reduce_population_count(mask)[0]


def _sload(ref, i):
    return ref[pl.ds(i, 16)][0]


def _sstore(ref, i, v):
    plsc.store_scatter(ref, [jnp.full((16,), i, jnp.int32)],
                       jnp.full((16,), v, jnp.int32), mask=_iota16() == 0)


def _body(f128_hbm, b_hbm, y_hbm, x_hbm, out_hbm,
          cbuf, rawl, wmap, staging, gbuf0, gbuf1, idxb0, idxb1,
          sem0, sem1, boff, pos):
    wid = lax.axis_index("s") * 2 + lax.axis_index("c")

    neg1 = jnp.full((16,), -1, jnp.int32)

    @pl.loop(0, CPT, step=16)
    def _(i):
        wmap[pl.ds(i, 16)] = neg1

    def a1_chunk(ci, count):
        base = ci * CHUNK
        pltpu.sync_copy(b_hbm.at[pl.ds(base, CHUNK)],
                        cbuf.at[pl.ds(0, CHUNK)])
        pltpu.sync_copy(y_hbm.at[pl.ds(base, CHUNK)],
                        cbuf.at[pl.ds(CHUNK, CHUNK)])
        pltpu.sync_copy(x_hbm.at[pl.ds(base, CHUNK)],
                        cbuf.at[pl.ds(2 * CHUNK, CHUNK)])

        def vloop(vi, cnt):
            off = vi * 16
            bv = cbuf[pl.ds(off, 16)]
            yv = cbuf[pl.ds(CHUNK + off, 16)]
            xv = cbuf[pl.ds(2 * CHUNK + off, 16)]
            cell = bv * (NYc * NXc) + yv * NXc + xv
            valid = lax.shift_right_logical(cell, LOG2_CPT) == wid
            local = lax.bitwise_and(cell, CPT - 1)
            p = base + off + _iota16()

            wv = plsc.load_gather(wmap, [local])
            m0 = valid & (p > wv)

            def wcond(m):
                return plsc.all_reduce_population_count(m)[0] > 0

            def wbody(m):
                plsc.store_scatter(wmap, [local], p, mask=m)
                w2 = plsc.load_gather(wmap, [local])
                return valid & (p > w2)

            lax.while_loop(wcond, wbody, m0)

            packed = lax.bitwise_or(local, lax.shift_left(p, LOG2_CPT))
            plsc.store_compressed(rawl.at[pl.ds(cnt, 16)], packed, mask=valid)
            return cnt + _popcount(valid)

        return lax.fori_loop(0, CHUNK // 16, vloop, count)

    with jax.named_scope("phaseA1"):
        na = lax.fori_loop(0, NCHUNK, a1_chunk, jnp.int32(0))

    @pl.loop(0, NREG)
    def _(r):
        pos[r] = 0

    def count_body(e, _):
        pk = _sload(rawl, e)
        local = lax.bitwise_and(pk, CPT - 1)
        p = lax.shift_right_logical(pk, LOG2_CPT)
        live_i = (_sload(wmap, local) == p).astype(jnp.int32)
        rg = lax.shift_right_logical(local, 9)
        pos[rg] += live_i
        _sstore(rawl, e, lax.bitwise_or(pk, lax.shift_left(live_i, 31)))
        return 0

    with jax.named_scope("phaseA2count"):
        lax.fori_loop(0, na, count_body, 0)

    def prefix_body(r, acc):
        c = pos[r]
        boff[r] = acc
        pos[r] = acc
        return acc + c

    nb = lax.fori_loop(0, NREG, prefix_body, jnp.int32(0))
    boff[NREG] = nb

    def place_body(e, _):
        pk = _sload(rawl, e)
        live = pk < 0
        pkc = lax.bitwise_and(pk, 0x7FFFFFFF)
        rg = lax.shift_right_logical(lax.bitwise_and(pkc, CPT - 1), 9)
        o = pos[rg]
        dest = jnp.where(live, o, DUMP)
        _sstore(wmap, dest, pkc)
        pos[rg] = o + live.astype(jnp.int32)
        return 0

    with jax.named_scope("phaseA2place"):
        lax.fori_loop(0, na, place_body, 0)

    zero16 = jnp.zeros((16,), jnp.float32)
    chanbase = (wid >> 3) * Cc
    yrowbase = lax.bitwise_and(wid, 7) * NREG

    @pl.loop(0, Cc)
    def _(c):
        @pl.loop(0, RCELLS, step=16)
        def _(j):
            staging[c, pl.ds(j, 16)] = zero16

    def fire(r, idx_s, gbuf_s, sem_s):
        @pl.when(r < NREG)
        def _():
            start = boff[r]
            end = boff[r + 1]
            for v in range(GROWS // 16):
                pk = wmap[pl.ds(start + v * 16, 16)]
                okm = (start + v * 16 + _iota16()) < end
                alt = lax.rem(start + v * 16 + _iota16(), Pc // 2)
                rid = jnp.where(okm,
                                lax.shift_right_logical(pk, LOG2_CPT + 1), alt)
                idx_s[pl.ds(v * 16, 16)] = rid
            pltpu.sync_copy(f128_hbm.at[idx_s], gbuf_s)

    def place_range(gbuf_s, lstart, n):
        def place(j, _):
            pkj = _sload(wmap, lstart + j)
            lr = lax.bitwise_and(pkj, RCELLS - 1)
            pj = lax.shift_right_logical(pkj, LOG2_CPT)
            half = lax.bitwise_and(pj, 1) * Cc
            lr_s = jnp.full((16,), lr, jnp.int32)
            for q in range(4):
                vals = gbuf_s[j, pl.ds(half + q * 16, 16)]
                plsc.store_scatter(staging, [q * 16 + _iota16(), lr_s], vals)
            return 0

        lax.fori_loop(0, n, place, 0)

    def process(r, idx_s, gbuf_s, sem_s):
        start = boff[r]
        end = boff[r + 1]
        n = end - start
        place_range(gbuf_s.at[pl.ds(0, GROWS)], start, jnp.minimum(n, GROWS))

        @pl.when(n > GROWS)
        def _():
            def chunk_body(ch, _):
                cstart = start + GROWS + ch * 16
                pk = wmap[pl.ds(cstart, 16)]
                okm = (cstart + _iota16()) < end
                rid = jnp.where(okm,
                                lax.shift_right_logical(pk, LOG2_CPT + 1), 0)
                idx_s[pl.ds(0, 16)] = rid
                pltpu.sync_copy(f128_hbm.at[idx_s.at[pl.ds(0, 16)]],
                                gbuf_s.at[pl.ds(0, 16)])
                k = jnp.minimum(jnp.int32(16), end - cstart)
                place_range(gbuf_s.at[pl.ds(0, 16)], cstart, k)
                return 0

            lax.fori_loop(0, (n - GROWS + 15) >> 4, chunk_body, 0)

        pltpu.sync_copy(staging,
                        out_hbm.at[pl.ds(chanbase, Cc),
                                   yrowbase + r,
                                   pl.ds(0, NXc)])

        def clean(e, _):
            pkj = _sload(wmap, e)
            lr = lax.bitwise_and(pkj, RCELLS - 1)
            lr_s = jnp.full((16,), lr, jnp.int32)
            for q in range(4):
                plsc.store_scatter(staging, [q * 16 + _iota16(), lr_s], zero16)
            return 0

        lax.fori_loop(start, end, clean, 0)

    with jax.named_scope("phaseB"):
        fire(jnp.int32(0), idxb0, gbuf0, sem0)

        def pair_body(rr, _):
            r0 = rr * 2
            fire(r0 + 1, idxb1, gbuf1, sem1)
            process(r0, idxb0, gbuf0, sem0)
            fire(r0 + 2, idxb0, gbuf0, sem0)
            process(r0 + 1, idxb1, gbuf1, sem1)
            return 0

        lax.fori_loop(0, NREG // 2, pair_body, 0)


@jax.jit
def kernel(pillar_features, voxel_coords):
    f128 = pillar_features.reshape(Pc // 2, 2 * Cc)
    cols = voxel_coords.T
    bcol = cols[0]
    ycol = cols[2]
    xcol = cols[3]

    mesh = plsc.VectorSubcoreMesh(core_axis_name="c", subcore_axis_name="s")
    run = pl.kernel(
        _body,
        out_type=jax.ShapeDtypeStruct((Bc * Cc, NYc, NXc), jnp.float32),
        mesh=mesh,
        scratch_types=[
            pltpu.VMEM((3 * CHUNK,), jnp.int32),
            pltpu.VMEM((Pc,), jnp.int32),
            pltpu.VMEM((CPT + 32,), jnp.int32),
            pltpu.VMEM((Cc, RCELLS), jnp.float32),
            pltpu.VMEM((GROWS, 2 * Cc), jnp.float32),
            pltpu.VMEM((GROWS, 2 * Cc), jnp.float32),
            pltpu.VMEM((GROWS,), jnp.int32),
            pltpu.VMEM((GROWS,), jnp.int32),
            pltpu.SemaphoreType.DMA,
            pltpu.SemaphoreType.DMA,
            pltpu.SMEM((NREG + 1,), jnp.int32),
            pltpu.SMEM((NREG,), jnp.int32),
        ],
        compiler_params=pltpu.CompilerParams(needs_layout_passes=False),
    )
    out_flat = run(f128, bcol, ycol, xcol)
    return out_flat.reshape(Bc, Cc, NYc, NXc)


# speedup vs baseline: 8.8556x; 1.2620x over previous
import functools

import jax
import jax.numpy as jnp
from jax import lax
from jax.experimental import pallas as pl
from jax.experimental.pallas import tpu as pltpu
from jax.experimental.pallas import tpu_sc as plsc

NXc, NYc, Cc, Bc, Pc = 512, 512, 64, 4, 40000
CPT = 32768
LOG2_CPT = 15
NREG = 128
RCELLS = NXc // 2
CHUNK = 2000
NCHUNK = Pc // CHUNK
DUMP = CPT
GROWS = 32
LOG2_RC = 8


def _iota16():
    return lax.iota(jnp.int32, 16)


def _popcount(mask):
    return plsc.all_reduce_population_count(mask)[0]


def _sload(ref, i):
    return ref[pl.ds(i, 16)][0]


def _sstore(ref, i, v):
    plsc.store_scatter(ref, [jnp.full((16,), i, jnp.int32)],
                       jnp.full((16,), v, jnp.int32), mask=_iota16() == 0)


def _body(f128_hbm, b_hbm, y_hbm, x_hbm, out_hbm,
          cbuf, rawl, wmap, stg0, stg1, gbuf0, gbuf1, idxb0, idxb1,
          semg0, semg1, semo0, semo1, boff, pos):
    wid = lax.axis_index("s") * 2 + lax.axis_index("c")

    neg1 = jnp.full((16,), -1, jnp.int32)

    @pl.loop(0, CPT, step=16)
    def _(i):
        wmap[pl.ds(i, 16)] = neg1

    def a1_chunk(ci, count):
        base = ci * CHUNK
        pltpu.sync_copy(b_hbm.at[pl.ds(base, CHUNK)],
                        cbuf.at[pl.ds(0, CHUNK)])
        pltpu.sync_copy(y_hbm.at[pl.ds(base, CHUNK)],
                        cbuf.at[pl.ds(CHUNK, CHUNK)])
        pltpu.sync_copy(x_hbm.at[pl.ds(base, CHUNK)],
                        cbuf.at[pl.ds(2 * CHUNK, CHUNK)])

        def vloop(vi, cnt):
            off = vi * 16
            bv = cbuf[pl.ds(off, 16)]
            yv = cbuf[pl.ds(CHUNK + off, 16)]
            xv = cbuf[pl.ds(2 * CHUNK + off, 16)]
            cell = bv * (NYc * NXc) + yv * NXc + xv
            valid = lax.shift_right_logical(cell, LOG2_CPT) == wid
            local = lax.bitwise_and(cell, CPT - 1)
            p = base + off + _iota16()

            wv = plsc.load_gather(wmap, [local])
            m0 = valid & (p > wv)

            def wcond(m):
                return plsc.all_reduce_population_count(m)[0] > 0

            def wbody(m):
                plsc.store_scatter(wmap, [local], p, mask=m)
                w2 = plsc.load_gather(wmap, [local])
                return valid & (p > w2)

            lax.while_loop(wcond, wbody, m0)

            packed = lax.bitwise_or(local, lax.shift_left(p, LOG2_CPT))
            plsc.store_compressed(rawl.at[pl.ds(cnt, 16)], packed, mask=valid)
            return cnt + _popcount(valid)

        return lax.fori_loop(0, CHUNK // 16, vloop, count)

    with jax.named_scope("phaseA1"):
        na = lax.fori_loop(0, NCHUNK, a1_chunk, jnp.int32(0))

    @pl.loop(0, NREG)
    def _(r):
        pos[r] = 0

    def count_body(e, _):
        pk = _sload(rawl, e)
        local = lax.bitwise_and(pk, CPT - 1)
        p = lax.shift_right_logical(pk, LOG2_CPT)
        live_i = (_sload(wmap, local) == p).astype(jnp.int32)
        rg = lax.shift_right_logical(local, LOG2_RC)
        pos[rg] += live_i
        _sstore(rawl, e, lax.bitwise_or(pk, lax.shift_left(live_i, 31)))
        return 0

    with jax.named_scope("phaseA2count"):
        lax.fori_loop(0, na, count_body, 0)

    def prefix_body(r, acc):
        c = pos[r]
        boff[r] = acc
        pos[r] = acc
        return acc + c

    nb = lax.fori_loop(0, NREG, prefix_body, jnp.int32(0))
    boff[NREG] = nb

    def place_body(e, _):
        pk = _sload(rawl, e)
        live = pk < 0
        pkc = lax.bitwise_and(pk, 0x7FFFFFFF)
        rg = lax.shift_right_logical(lax.bitwise_and(pkc, CPT - 1), LOG2_RC)
        o = pos[rg]
        dest = jnp.where(live, o, DUMP)
        _sstore(wmap, dest, pkc)
        pos[rg] = o + live.astype(jnp.int32)
        return 0

    with jax.named_scope("phaseA2place"):
        lax.fori_loop(0, na, place_body, 0)

    zero16 = jnp.zeros((16,), jnp.float32)
    chanbase = (wid >> 3) * Cc
    yrowbase = lax.bitwise_and(wid, 7) * (NREG // 2)

    for stg in (stg0, stg1):
        @pl.loop(0, Cc)
        def _(c):
            @pl.loop(0, RCELLS, step=16)
            def _(j):
                stg[c, pl.ds(j, 16)] = zero16

    def out_dst(r):
        return out_hbm.at[pl.ds(chanbase, Cc),
                          yrowbase + lax.shift_right_logical(r, 1),
                          pl.ds(lax.bitwise_and(r, 1) * RCELLS, RCELLS)]

    def fire_gather(r, idx_s, gbuf_s, sem_g):
        @pl.when(r < NREG)
        def _():
            start = boff[r]
            end = boff[r + 1]
            for v in range(GROWS // 16):
                pk = wmap[pl.ds(start + v * 16, 16)]
                okm = (start + v * 16 + _iota16()) < end
                alt = lax.rem(start + v * 16 + _iota16(), Pc // 2)
                rid = jnp.where(okm,
                                lax.shift_right_logical(pk, LOG2_CPT + 1), alt)
                idx_s[pl.ds(v * 16, 16)] = rid
            pltpu.async_copy(f128_hbm.at[idx_s], gbuf_s, sem_g)

    def place_range(gbuf_s, stg, lstart, n):
        def place(j, _):
            pkj = _sload(wmap, lstart + j)
            lr = lax.bitwise_and(pkj, RCELLS - 1)
            pj = lax.shift_right_logical(pkj, LOG2_CPT)
            half = lax.bitwise_and(pj, 1) * Cc
            lr_s = jnp.full((16,), lr, jnp.int32)
            for q in range(4):
                vals = gbuf_s[j, pl.ds(half + q * 16, 16)]
                plsc.store_scatter(stg, [q * 16 + _iota16(), lr_s], vals)
            return 0

        lax.fori_loop(0, n, place, 0)

    def clean_range(r, stg):
        def clean(e, _):
            pkj = _sload(wmap, e)
            lr = lax.bitwise_and(pkj, RCELLS - 1)
            lr_s = jnp.full((16,), lr, jnp.int32)
            for q in range(4):
                plsc.store_scatter(stg, [q * 16 + _iota16(), lr_s], zero16)
            return 0

        lax.fori_loop(boff[r], boff[r + 1], clean, 0)

    def step(r, idx_s, gbuf_s, sem_g, stg, sem_o):
        @pl.when(r >= 2)
        def _():
            pltpu.make_async_copy(stg, out_dst(r), sem_o).wait()
            clean_range(r - 2, stg)

        start = boff[r]
        end = boff[r + 1]
        n = end - start
        pltpu.make_async_copy(f128_hbm.at[idx_s], gbuf_s, sem_g).wait()
        place_range(gbuf_s.at[pl.ds(0, GROWS)], stg, start,
                    jnp.minimum(n, GROWS))

        @pl.when(n > GROWS)
        def _():
            def chunk_body(ch, _):
                cstart = start + GROWS + ch * 16
                pk = wmap[pl.ds(cstart, 16)]
                okm = (cstart + _iota16()) < end
                alt = lax.rem(cstart + _iota16(), Pc // 2)
                rid = jnp.where(okm,
                                lax.shift_right_logical(pk, LOG2_CPT + 1), alt)
                idx_s[pl.ds(0, 16)] = rid
                pltpu.sync_copy(f128_hbm.at[idx_s.at[pl.ds(0, 16)]],
                                gbuf_s.at[pl.ds(0, 16)])
                k = jnp.minimum(jnp.int32(16), end - cstart)
                place_range(gbuf_s.at[pl.ds(0, 16)], stg, cstart, k)
                return 0

            lax.fori_loop(0, (n - GROWS + 15) >> 4, chunk_body, 0)

        pltpu.async_copy(stg, out_dst(r), sem_o)
        fire_gather(r + 2, idx_s, gbuf_s, sem_g)

    with jax.named_scope("phaseB"):
        fire_gather(jnp.int32(0), idxb0, gbuf0, semg0)
        fire_gather(jnp.int32(1), idxb1, gbuf1, semg1)

        def pair_body(rr, _):
            r0 = rr * 2
            step(r0, idxb0, gbuf0, semg0, stg0, semo0)
            step(r0 + 1, idxb1, gbuf1, semg1, stg1, semo1)
            return 0

        lax.fori_loop(0, NREG // 2, pair_body, 0)
        pltpu.make_async_copy(stg0, out_dst(jnp.int32(NREG - 2)), semo0).wait()
        pltpu.make_async_copy(stg1, out_dst(jnp.int32(NREG - 1)), semo1).wait()


@jax.jit
def kernel(pillar_features, voxel_coords):
    f128 = pillar_features.reshape(Pc // 2, 2 * Cc)
    cols = voxel_coords.T
    bcol = cols[0]
    ycol = cols[2]
    xcol = cols[3]

    mesh = plsc.VectorSubcoreMesh(core_axis_name="c", subcore_axis_name="s")
    run = pl.kernel(
        _body,
        out_type=jax.ShapeDtypeStruct((Bc * Cc, NYc, NXc), jnp.float32),
        mesh=mesh,
        scratch_types=[
            pltpu.VMEM((3 * CHUNK,), jnp.int32),
            pltpu.VMEM((Pc,), jnp.int32),
            pltpu.VMEM((CPT + 32,), jnp.int32),
            pltpu.VMEM((Cc, RCELLS), jnp.float32),
            pltpu.VMEM((Cc, RCELLS), jnp.float32),
            pltpu.VMEM((GROWS, 2 * Cc), jnp.float32),
            pltpu.VMEM((GROWS, 2 * Cc), jnp.float32),
            pltpu.VMEM((GROWS,), jnp.int32),
            pltpu.VMEM((GROWS,), jnp.int32),
            pltpu.SemaphoreType.DMA,
            pltpu.SemaphoreType.DMA,
            pltpu.SemaphoreType.DMA,
            pltpu.SemaphoreType.DMA,
            pltpu.SMEM((NREG + 1,), jnp.int32),
            pltpu.SMEM((NREG,), jnp.int32),
        ],
        compiler_params=pltpu.CompilerParams(needs_layout_passes=False),
    )
    out_flat = run(f128, bcol, ycol, xcol)
    return out_flat.reshape(Bc, Cc, NYc, NXc)


# speedup vs baseline: 9.5286x; 1.0760x over previous
import functools

import jax
import jax.numpy as jnp
from jax import lax
from jax.experimental import pallas as pl
from jax.experimental.pallas import tpu as pltpu
from jax.experimental.pallas import tpu_sc as plsc

NXc, NYc, Cc, Bc, Pc = 512, 512, 64, 4, 40000
CPT = 32768
LOG2_CPT = 15
NREG = 128
RCELLS = NXc // 2
CHUNK = 2048
PPAD = 40960
NCHUNK = PPAD // CHUNK
DUMP = CPT
GROWS = 32
LOG2_RC = 8


def _iota16():
    return lax.iota(jnp.int32, 16)


def _popcount(mask):
    return plsc.all_reduce_population_count(mask)[0]


def _sload(ref, i):
    return ref[pl.ds(i, 16)][0]


def _sstore(ref, i, v):
    plsc.store_scatter(ref, [jnp.full((16,), i, jnp.int32)],
                       jnp.full((16,), v, jnp.int32), mask=_iota16() == 0)


def _body(f128_hbm, c3_hbm, out_hbm,
          cbufa, cbufb, rawl, wmap, stg0, stg1, gbuf0, gbuf1, idxb0, idxb1,
          semca, semcb, semg0, semg1, semo0, semo1, boff, pos):
    wid = lax.axis_index("s") * 2 + lax.axis_index("c")

    neg1 = jnp.full((16,), -1, jnp.int32)

    @pl.loop(0, CPT, step=16)
    def _(i):
        wmap[pl.ds(i, 16)] = neg1

    def fire_chunk(ci, cb, sem):
        @pl.when(ci < NCHUNK)
        def _():
            pltpu.async_copy(
                c3_hbm.at[pl.ds(0, 3), pl.ds(ci * CHUNK, CHUNK)], cb, sem)

    def a1_chunk(ci, cbuf, sem, count):
        base = ci * CHUNK
        pltpu.make_async_copy(
            c3_hbm.at[pl.ds(0, 3), pl.ds(base, CHUNK)], cbuf, sem).wait()

        def vloop(vi, cnt):
            off = vi * 16
            bv = cbuf[0, pl.ds(off, 16)]
            yv = cbuf[1, pl.ds(off, 16)]
            xv = cbuf[2, pl.ds(off, 16)]
            cell = bv * (NYc * NXc) + yv * NXc + xv
            valid = lax.shift_right_logical(cell, LOG2_CPT) == wid
            local = lax.bitwise_and(cell, CPT - 1)
            p = base + off + _iota16()

            wv = plsc.load_gather(wmap, [local])
            m0 = valid & (p > wv)

            def wcond(m):
                return plsc.all_reduce_population_count(m)[0] > 0

            def wbody(m):
                plsc.store_scatter(wmap, [local], p, mask=m)
                w2 = plsc.load_gather(wmap, [local])
                return valid & (p > w2)

            lax.while_loop(wcond, wbody, m0)

            packed = lax.bitwise_or(local, lax.shift_left(p, LOG2_CPT))
            plsc.store_compressed(rawl.at[pl.ds(cnt, 16)], packed, mask=valid)
            return cnt + _popcount(valid)

        cnt2 = lax.fori_loop(0, CHUNK // 16, vloop, count)
        return cnt2

    with jax.named_scope("phaseA1"):
        fire_chunk(jnp.int32(0), cbufa, semca)
        fire_chunk(jnp.int32(1), cbufb, semcb)

        def a1_pair(pp, count):
            ci = pp * 2
            count = a1_chunk(ci, cbufa, semca, count)
            fire_chunk(ci + 2, cbufa, semca)
            count = a1_chunk(ci + 1, cbufb, semcb, count)
            fire_chunk(ci + 3, cbufb, semcb)
            return count

        na = lax.fori_loop(0, NCHUNK // 2, a1_pair, jnp.int32(0))

    @pl.loop(0, NREG)
    def _(r):
        pos[r] = 0

    def count_body(e, _):
        pk = _sload(rawl, e)
        local = lax.bitwise_and(pk, CPT - 1)
        p = lax.shift_right_logical(pk, LOG2_CPT)
        live_i = (_sload(wmap, local) == p).astype(jnp.int32)
        rg = lax.shift_right_logical(local, LOG2_RC)
        pos[rg] += live_i
        _sstore(rawl, e, lax.bitwise_or(pk, lax.shift_left(live_i, 31)))
        return 0

    with jax.named_scope("phaseA2count"):
        lax.fori_loop(0, na, count_body, 0)

    def prefix_body(r, acc):
        c = pos[r]
        boff[r] = acc
        pos[r] = acc
        return acc + c

    nb = lax.fori_loop(0, NREG, prefix_body, jnp.int32(0))
    boff[NREG] = nb

    def place_body(e, _):
        pk = _sload(rawl, e)
        live = pk < 0
        pkc = lax.bitwise_and(pk, 0x7FFFFFFF)
        rg = lax.shift_right_logical(lax.bitwise_and(pkc, CPT - 1), LOG2_RC)
        o = pos[rg]
        dest = jnp.where(live, o, DUMP)
        _sstore(wmap, dest, pkc)
        pos[rg] = o + live.astype(jnp.int32)
        return 0

    with jax.named_scope("phaseA2place"):
        lax.fori_loop(0, na, place_body, 0)

    zero16 = jnp.zeros((16,), jnp.float32)
    chanbase = (wid >> 3) * Cc
    yrowbase = lax.bitwise_and(wid, 7) * (NREG // 2)

    for stg in (stg0, stg1):
        @pl.loop(0, Cc)
        def _(c):
            @pl.loop(0, RCELLS, step=16)
            def _(j):
                stg[c, pl.ds(j, 16)] = zero16

    def out_dst(r):
        return out_hbm.at[pl.ds(chanbase, Cc),
                          yrowbase + lax.shift_right_logical(r, 1),
                          pl.ds(lax.bitwise_and(r, 1) * RCELLS, RCELLS)]

    def fire_gather(r, idx_s, gbuf_s, sem_g):
        @pl.when(r < NREG)
        def _():
            start = boff[r]
            end = boff[r + 1]
            for v in range(GROWS // 16):
                pk = wmap[pl.ds(start + v * 16, 16)]
                okm = (start + v * 16 + _iota16()) < end
                alt = lax.rem(start + v * 16 + _iota16(), Pc // 2)
                rid = jnp.where(okm,
                                lax.shift_right_logical(pk, LOG2_CPT + 1), alt)
                idx_s[pl.ds(v * 16, 16)] = rid
            pltpu.async_copy(f128_hbm.at[idx_s], gbuf_s, sem_g)

    def place_range(gbuf_s, stg, lstart, n):
        def place(j, _):
            pkj = _sload(wmap, lstart + j)
            lr = lax.bitwise_and(pkj, RCELLS - 1)
            pj = lax.shift_right_logical(pkj, LOG2_CPT)
            half = lax.bitwise_and(pj, 1) * Cc
            lr_s = jnp.full((16,), lr, jnp.int32)
            for q in range(4):
                vals = gbuf_s[j, pl.ds(half + q * 16, 16)]
                plsc.store_scatter(stg, [q * 16 + _iota16(), lr_s], vals)
            return 0

        lax.fori_loop(0, n, place, 0)

    def clean_range(r, stg):
        def clean(e, _):
            pkj = _sload(wmap, e)
            lr = lax.bitwise_and(pkj, RCELLS - 1)
            lr_s = jnp.full((16,), lr, jnp.int32)
            for q in range(4):
                plsc.store_scatter(stg, [q * 16 + _iota16(), lr_s], zero16)
            return 0

        lax.fori_loop(boff[r], boff[r + 1], clean, 0)

    def step(r, idx_s, gbuf_s, sem_g, stg, sem_o):
        @pl.when(r >= 2)
        def _():
            pltpu.make_async_copy(stg, out_dst(r), sem_o).wait()
            clean_range(r - 2, stg)

        start = boff[r]
        end = boff[r + 1]
        n = end - start
        pltpu.make_async_copy(f128_hbm.at[idx_s], gbuf_s, sem_g).wait()
        place_range(gbuf_s.at[pl.ds(0, GROWS)], stg, start,
                    jnp.minimum(n, GROWS))

        @pl.when(n > GROWS)
        def _():
            def chunk_body(ch, _):
                cstart = start + GROWS + ch * 16
                pk = wmap[pl.ds(cstart, 16)]
                okm = (cstart + _iota16()) < end
                alt = lax.rem(cstart + _iota16(), Pc // 2)
                rid = jnp.where(okm,
                                lax.shift_right_logical(pk, LOG2_CPT + 1), alt)
                idx_s[pl.ds(0, 16)] = rid
                pltpu.sync_copy(f128_hbm.at[idx_s.at[pl.ds(0, 16)]],
                                gbuf_s.at[pl.ds(0, 16)])
                k = jnp.minimum(jnp.int32(16), end - cstart)
                place_range(gbuf_s.at[pl.ds(0, 16)], stg, cstart, k)
                return 0

            lax.fori_loop(0, (n - GROWS + 15) >> 4, chunk_body, 0)

        pltpu.async_copy(stg, out_dst(r), sem_o)
        fire_gather(r + 2, idx_s, gbuf_s, sem_g)

    with jax.named_scope("phaseB"):
        fire_gather(jnp.int32(0), idxb0, gbuf0, semg0)
        fire_gather(jnp.int32(1), idxb1, gbuf1, semg1)

        def pair_body(rr, _):
            r0 = rr * 2
            step(r0, idxb0, gbuf0, semg0, stg0, semo0)
            step(r0 + 1, idxb1, gbuf1, semg1, stg1, semo1)
            return 0

        lax.fori_loop(0, NREG // 2, pair_body, 0)
        pltpu.make_async_copy(stg0, out_dst(jnp.int32(NREG - 2)), semo0).wait()
        pltpu.make_async_copy(stg1, out_dst(jnp.int32(NREG - 1)), semo1).wait()


@jax.jit
def kernel(pillar_features, voxel_coords):
    f128 = pillar_features.reshape(Pc // 2, 2 * Cc)
    cols = voxel_coords.T
    c3 = jnp.stack((cols[0], cols[2], cols[3]))
    c3 = jnp.concatenate(
        [c3, jnp.full((3, PPAD - Pc), Bc, jnp.int32)], axis=1)

    mesh = plsc.VectorSubcoreMesh(core_axis_name="c", subcore_axis_name="s")
    run = pl.kernel(
        _body,
        out_type=jax.ShapeDtypeStruct((Bc * Cc, NYc, NXc), jnp.float32),
        mesh=mesh,
        scratch_types=[
            pltpu.VMEM((3, CHUNK), jnp.int32),
            pltpu.VMEM((3, CHUNK), jnp.int32),
            pltpu.VMEM((Pc,), jnp.int32),
            pltpu.VMEM((CPT + 32,), jnp.int32),
            pltpu.VMEM((Cc, RCELLS), jnp.float32),
            pltpu.VMEM((Cc, RCELLS), jnp.float32),
            pltpu.VMEM((GROWS, 2 * Cc), jnp.float32),
            pltpu.VMEM((GROWS, 2 * Cc), jnp.float32),
            pltpu.VMEM((GROWS,), jnp.int32),
            pltpu.VMEM((GROWS,), jnp.int32),
            pltpu.SemaphoreType.DMA,
            pltpu.SemaphoreType.DMA,
            pltpu.SemaphoreType.DMA,
            pltpu.SemaphoreType.DMA,
            pltpu.SemaphoreType.DMA,
            pltpu.SemaphoreType.DMA,
            pltpu.SMEM((NREG + 1,), jnp.int32),
            pltpu.SMEM((NREG,), jnp.int32),
        ],
        compiler_params=pltpu.CompilerParams(needs_layout_passes=False),
    )
    out_flat = run(f128, c3)
    return out_flat.reshape(Bc, Cc, NYc, NXc)
